# Initial kernel scaffold; baseline (speedup 1.0000x reference)
import functools

import jax
import jax.numpy as jnp
from jax import lax
from jax.experimental import pallas as pl
from jax.experimental.pallas import tpu as pltpu
from jax.experimental.pallas import tpu_sc as plsc

N_NODES = 10000
N_EDGES = 320000
D = 128
L = 4
R_CUTOFF = 8.0

NC = 2
NS = 16
NW = NC * NS
CH = 128
K = -(-N_EDGES // (NW * CH))
E_PAD = NW * K * CH
ROWS_SUB = N_NODES // NS

NBLK = 1000
GRID_N = N_NODES // NBLK
EBLK = 1024
GRID_E = E_PAD // EBLK

_f32 = jnp.float32


def _embed_body(an_ref, emb_ref, w_ref, b_ref, x0_ref, h0_ref):
    an = an_ref[...]
    ids = lax.broadcasted_iota(_f32, (1, 128), 1)
    oh = (an == ids).astype(_f32)
    x0 = jnp.dot(oh, emb_ref[...], preferred_element_type=_f32)
    x0_ref[...] = x0
    h0_ref[...] = jnp.dot(x0, w_ref[...], preferred_element_type=_f32) + b_ref[...]


_embed_call = pl.pallas_call(
    _embed_body,
    grid=(GRID_N,),
    in_specs=[
        pl.BlockSpec((NBLK, 1), lambda i: (i, 0)),
        pl.BlockSpec((128, D), lambda i: (0, 0)),
        pl.BlockSpec((D, D), lambda i: (0, 0)),
        pl.BlockSpec((1, D), lambda i: (0, 0)),
    ],
    out_specs=[
        pl.BlockSpec((NBLK, D), lambda i: (i, 0)),
        pl.BlockSpec((NBLK, D), lambda i: (i, 0)),
    ],
    out_shape=[
        jax.ShapeDtypeStruct((N_NODES, D), _f32),
        jax.ShapeDtypeStruct((N_NODES, D), _f32),
    ],
)


def _filt_body(r_ref, w_ref, b_ref, o0, o1, o2, o3):
    i = pl.program_id(0)
    r = r_ref[...]
    d = jnp.sqrt(jnp.sum(r * r, axis=1, keepdims=True))
    step = R_CUTOFF / (D - 1)
    centers = lax.broadcasted_iota(_f32, (1, D), 1) * step
    gamma = 1.0 / (step * step)
    rbf = jnp.exp(-gamma * (d - centers) ** 2)
    f = jnp.dot(rbf, w_ref[...], preferred_element_type=_f32) + b_ref[...]
    rowid = i * EBLK + lax.broadcasted_iota(jnp.int32, (EBLK, 1), 0)
    f = jnp.where(rowid < N_EDGES, f, 0.0)
    o0[...] = f[:, 0:D]
    o1[...] = f[:, D:2 * D]
    o2[...] = f[:, 2 * D:3 * D]
    o3[...] = f[:, 3 * D:4 * D]


_filt_call = pl.pallas_call(
    _filt_body,
    grid=(GRID_E,),
    in_specs=[
        pl.BlockSpec((EBLK, 8), lambda i: (i, 0)),
        pl.BlockSpec((D, L * D), lambda i: (0, 0)),
        pl.BlockSpec((1, L * D), lambda i: (0, 0)),
    ],
    out_specs=[pl.BlockSpec((EBLK, D), lambda i: (i, 0)) for _ in range(L)],
    out_shape=[jax.ShapeDtypeStruct((E_PAD, D), _f32) for _ in range(L)],
)


def _silu(x):
    return x * jax.nn.sigmoid(x)


def _layer_tail(p_ref, x_ref, x0_ref, postw, postb, f1w, f1b, f2w, f2b):
    agg = p_ref[0] + p_ref[1]
    t = jnp.dot(agg, postw[...], preferred_element_type=_f32) + postb[...]
    x1 = _silu(t) + x_ref[...]
    u = _silu(jnp.dot(x1, f1w[...], preferred_element_type=_f32) + f1b[...])
    ff = jnp.dot(u, f2w[...], preferred_element_type=_f32) + f2b[...]
    return ff + x1 + x0_ref[...]


def _layer_body(p_ref, x_ref, x0_ref, postw, postb, f1w, f1b, f2w, f2b,
                prew, preb, xo_ref, ho_ref):
    x2 = _layer_tail(p_ref, x_ref, x0_ref, postw, postb, f1w, f1b, f2w, f2b)
    xo_ref[...] = x2
    ho_ref[...] = jnp.dot(x2, prew[...], preferred_element_type=_f32) + preb[...]


_W_SPEC = pl.BlockSpec((D, D), lambda i: (0, 0))
_B_SPEC = pl.BlockSpec((1, D), lambda i: (0, 0))
_X_SPEC = pl.BlockSpec((NBLK, D), lambda i: (i, 0))
_P_SPEC = pl.BlockSpec((NC, NBLK, D), lambda i: (0, i, 0))

_layer_call = pl.pallas_call(
    _layer_body,
    grid=(GRID_N,),
    in_specs=[_P_SPEC, _X_SPEC, _X_SPEC,
              _W_SPEC, _B_SPEC, _W_SPEC, _B_SPEC, _W_SPEC, _B_SPEC,
              _W_SPEC, _B_SPEC],
    out_specs=[_X_SPEC, _X_SPEC],
    out_shape=[
        jax.ShapeDtypeStruct((N_NODES, D), _f32),
        jax.ShapeDtypeStruct((N_NODES, D), _f32),
    ],
)


def _final_body(p_ref, x_ref, x0_ref, postw, postb, f1w, f1b, f2w, f2b,
                fcw_ref, fcb_ref, out_ref):
    i = pl.program_id(0)
    x2 = _layer_tail(p_ref, x_ref, x0_ref, postw, postb, f1w, f1b, f2w, f2b)
    e = jnp.dot(x2, fcw_ref[...], preferred_element_type=_f32)

    @pl.when(i == 0)
    def _():
        out_ref[0, 0] = 0.0

    out_ref[0, 0] += jnp.sum(e)

    @pl.when(i == GRID_N - 1)
    def _():
        out_ref[0, 0] = out_ref[0, 0] / N_NODES + fcb_ref[0, 0]


_final_call = pl.pallas_call(
    _final_body,
    grid=(GRID_N,),
    in_specs=[_P_SPEC, _X_SPEC, _X_SPEC,
              _W_SPEC, _B_SPEC, _W_SPEC, _B_SPEC, _W_SPEC, _B_SPEC,
              pl.BlockSpec((D, 1), lambda i: (0, 0)),
              pl.BlockSpec(memory_space=pltpu.SMEM)],
    out_specs=pl.BlockSpec(memory_space=pltpu.SMEM),
    out_shape=jax.ShapeDtypeStruct((1, 1), _f32),
)


def _sc_msg_body(h_hbm, filt_hbm, src_hbm, dst_hbm, zeros_hbm, out_hbm,
                 src_v, dst_v, filt_v, hrows_v, agg_sh, sem_f, sem_g):
    c = lax.axis_index("c")
    s = lax.axis_index("s")
    wid = c * NS + s

    pltpu.sync_copy(zeros_hbm, agg_sh.at[pl.ds(s * ROWS_SUB, ROWS_SUB)])
    pltpu.sync_copy(src_hbm.at[wid], src_v)
    pltpu.sync_copy(dst_hbm.at[wid], dst_v)
    plsc.subcore_barrier()

    def chunk(j, carry):
        base = (wid * K + j) * CH
        pltpu.async_copy(filt_hbm.at[pl.ds(base, CH)], filt_v, sem_f).wait()
        pltpu.async_copy(h_hbm.at[src_v.at[j]], hrows_v, sem_g).wait()

        def row(rr, cc):
            for q in range(8):
                sl = pl.ds(q * 16, 16)
                hrows_v[rr, sl] = hrows_v[rr, sl] * filt_v[rr, sl]
            return cc

        lax.fori_loop(0, CH, row, 0)
        pltpu.sync_copy(hrows_v, agg_sh.at[dst_v.at[j]], add=True)
        return carry

    lax.fori_loop(0, K, chunk, 0)
    plsc.subcore_barrier()
    pltpu.sync_copy(agg_sh.at[pl.ds(s * ROWS_SUB, ROWS_SUB)],
                    out_hbm.at[c, pl.ds(s * ROWS_SUB, ROWS_SUB)])


_sc_msg_call = pl.kernel(
    _sc_msg_body,
    out_type=jax.ShapeDtypeStruct((NC, N_NODES, D), _f32),
    mesh=plsc.VectorSubcoreMesh(core_axis_name="c", subcore_axis_name="s"),
    scratch_types=[
        pltpu.VMEM((K, CH), jnp.int32),
        pltpu.VMEM((K, CH), jnp.int32),
        pltpu.VMEM((CH, D), _f32),
        pltpu.VMEM((CH, D), _f32),
        pltpu.VMEM_SHARED((N_NODES, D), _f32),
        pltpu.SemaphoreType.DMA,
        pltpu.SemaphoreType.DMA,
    ],
)


def kernel(atomic_number, edge_index, r, atom_emb, pre_w, pre_b, filt_w, filt_b,
           post_w, post_b, ff_w1, ff_b1, ff_w2, ff_b2, fc_w, fc_b):
    an = atomic_number.astype(_f32).reshape(N_NODES, 1)
    emb_p = jnp.pad(atom_emb, ((0, 128 - atom_emb.shape[0]), (0, 0)))
    pre_wT = jnp.transpose(pre_w, (0, 2, 1))
    post_wT = jnp.transpose(post_w, (0, 2, 1))
    ff_w1T = jnp.transpose(ff_w1, (0, 2, 1))
    ff_w2T = jnp.transpose(ff_w2, (0, 2, 1))
    filt_wcat = jnp.concatenate([filt_w[i].T for i in range(L)], axis=1)
    filt_bcat = filt_b.reshape(1, L * D)
    fc_wT = fc_w.T
    fc_b2d = fc_b.reshape(1, 1)

    pad = E_PAD - N_EDGES
    src3 = jnp.pad(edge_index[0].astype(jnp.int32), (0, pad)).reshape(NW, K, CH)
    dst3 = jnp.pad(edge_index[1].astype(jnp.int32), (0, pad)).reshape(NW, K, CH)
    r8 = jnp.pad(r, ((0, pad), (0, 5)))
    zeros = jnp.zeros((ROWS_SUB, D), _f32)

    x0, h = _embed_call(an, emb_p, pre_wT[0], pre_b[0].reshape(1, D))
    filts = _filt_call(r8, filt_wcat, filt_bcat)

    x = x0
    out = None
    for i in range(L):
        p = _sc_msg_call(h, filts[i], src3, dst3, zeros)
        if i < L - 1:
            x, h = _layer_call(p, x, x0,
                               post_wT[i], post_b[i].reshape(1, D),
                               ff_w1T[i], ff_b1[i].reshape(1, D),
                               ff_w2T[i], ff_b2[i].reshape(1, D),
                               pre_wT[i + 1], pre_b[i + 1].reshape(1, D))
        else:
            out = _final_call(p, x, x0,
                              post_wT[i], post_b[i].reshape(1, D),
                              ff_w1T[i], ff_b1[i].reshape(1, D),
                              ff_w2T[i], ff_b2[i].reshape(1, D),
                              fc_wT, fc_b2d)
    return out[0, 0]


# speedup vs baseline: 2.4456x; 2.4456x over previous
import functools

import jax
import jax.numpy as jnp
from jax import lax
from jax.experimental import pallas as pl
from jax.experimental.pallas import tpu as pltpu
from jax.experimental.pallas import tpu_sc as plsc

N_NODES = 10000
N_EDGES = 320000
D = 128
L = 4
R_CUTOFF = 8.0

NC = 2
NS = 16
NW = NC * NS
CH = 64
K = (-(-N_EDGES // (NW * CH)) + 7) // 8 * 8
K0 = K
K1 = 2 * K - K0
E_PAD = NS * (K0 + K1) * CH
N_PAD = 10240
ROWS_SUB = N_PAD // NS
CBYTES = CH * D * 4
QN = CH // 16

NBLK = 1000
GRID_N = N_NODES // NBLK
EBLK = 1024
GRID_E = E_PAD // EBLK

_f32 = jnp.float32


def _embed_body(an_ref, emb_ref, w_ref, b_ref, x0_ref, h0_ref):
    an = an_ref[...]
    ids = lax.broadcasted_iota(jnp.int32, (1, 128), 1).astype(_f32)
    oh = (an == ids).astype(_f32)
    x0 = jnp.dot(oh, emb_ref[...], preferred_element_type=_f32)
    x0_ref[...] = x0
    h0_ref[...] = jnp.dot(x0, w_ref[...], preferred_element_type=_f32) + b_ref[...]


_embed_call = pl.pallas_call(
    _embed_body,
    grid=(GRID_N,),
    in_specs=[
        pl.BlockSpec((NBLK, 1), lambda i: (i, 0)),
        pl.BlockSpec((128, D), lambda i: (0, 0)),
        pl.BlockSpec((D, D), lambda i: (0, 0)),
        pl.BlockSpec((1, D), lambda i: (0, 0)),
    ],
    out_specs=[
        pl.BlockSpec((NBLK, D), lambda i: (i, 0)),
        pl.BlockSpec((NBLK, D), lambda i: (i, 0)),
    ],
    out_shape=[
        jax.ShapeDtypeStruct((N_NODES, D), _f32),
        jax.ShapeDtypeStruct((N_NODES, D), _f32),
    ],
)


def _filt_body(r_ref, w_ref, b_ref, o0, o1, o2, o3):
    i = pl.program_id(0)
    r = r_ref[...]
    d = jnp.sqrt(jnp.sum(r * r, axis=1, keepdims=True))
    step = R_CUTOFF / (D - 1)
    centers = lax.broadcasted_iota(jnp.int32, (1, D), 1).astype(_f32) * step
    gamma = 1.0 / (step * step)
    rbf = jnp.exp(-gamma * (d - centers) ** 2)
    f = jnp.dot(rbf, w_ref[...], preferred_element_type=_f32) + b_ref[...]
    rowid = i * EBLK + lax.broadcasted_iota(jnp.int32, (EBLK, 1), 0)
    f = jnp.where(rowid < N_EDGES, f, 0.0)
    for l, o in enumerate((o0, o1, o2, o3)):
        fl = f[:, l * D:(l + 1) * D]
        bits = jax.lax.bitcast_convert_type(fl, jnp.int32)
        r16 = jax.lax.shift_right_logical(bits + 0x8000, 16)
        lo = jnp.bitwise_and(r16[:, :D // 2], 0xFFFF)
        hi = jax.lax.shift_left(r16[:, D // 2:], 16)
        o[...] = jnp.bitwise_or(lo, hi)


_filt_call = pl.pallas_call(
    _filt_body,
    grid=(GRID_E,),
    in_specs=[
        pl.BlockSpec((EBLK, 8), lambda i: (i, 0)),
        pl.BlockSpec((D, L * D), lambda i: (0, 0)),
        pl.BlockSpec((1, L * D), lambda i: (0, 0)),
    ],
    out_specs=[pl.BlockSpec((EBLK, D // 2), lambda i: (i, 0)) for _ in range(L)],
    out_shape=[jax.ShapeDtypeStruct((E_PAD, D // 2), jnp.int32) for _ in range(L)],
)


def _silu(x):
    return x * jax.nn.sigmoid(x)


def _layer_tail(p_ref, x_ref, x0_ref, postw, postb, f1w, f1b, f2w, f2b):
    agg = p_ref[0] + p_ref[1]
    t = jnp.dot(agg, postw[...], preferred_element_type=_f32) + postb[...]
    x1 = _silu(t) + x_ref[...]
    u = _silu(jnp.dot(x1, f1w[...], preferred_element_type=_f32) + f1b[...])
    ff = jnp.dot(u, f2w[...], preferred_element_type=_f32) + f2b[...]
    return ff + x1 + x0_ref[...]


def _layer_body(p_ref, x_ref, x0_ref, postw, postb, f1w, f1b, f2w, f2b,
                prew, preb, xo_ref, ho_ref):
    x2 = _layer_tail(p_ref, x_ref, x0_ref, postw, postb, f1w, f1b, f2w, f2b)
    xo_ref[...] = x2
    ho_ref[...] = jnp.dot(x2, prew[...], preferred_element_type=_f32) + preb[...]


_W_SPEC = pl.BlockSpec((D, D), lambda i: (0, 0))
_B_SPEC = pl.BlockSpec((1, D), lambda i: (0, 0))
_X_SPEC = pl.BlockSpec((NBLK, D), lambda i: (i, 0))
_P_SPEC = pl.BlockSpec((NC, NBLK, D), lambda i: (0, i, 0))

_layer_call = pl.pallas_call(
    _layer_body,
    grid=(GRID_N,),
    in_specs=[_P_SPEC, _X_SPEC, _X_SPEC,
              _W_SPEC, _B_SPEC, _W_SPEC, _B_SPEC, _W_SPEC, _B_SPEC,
              _W_SPEC, _B_SPEC],
    out_specs=[_X_SPEC, _X_SPEC],
    out_shape=[
        jax.ShapeDtypeStruct((N_NODES, D), _f32),
        jax.ShapeDtypeStruct((N_NODES, D), _f32),
    ],
)


def _final_body(p_ref, x_ref, x0_ref, postw, postb, f1w, f1b, f2w, f2b,
                fcw_ref, fcb_ref, out_ref):
    i = pl.program_id(0)
    x2 = _layer_tail(p_ref, x_ref, x0_ref, postw, postb, f1w, f1b, f2w, f2b)
    e = jnp.dot(x2, fcw_ref[...], preferred_element_type=_f32)

    @pl.when(i == 0)
    def _():
        out_ref[0, 0] = 0.0

    out_ref[0, 0] += jnp.sum(e)

    @pl.when(i == GRID_N - 1)
    def _():
        out_ref[0, 0] = out_ref[0, 0] / N_NODES + fcb_ref[0, 0]


_final_call = pl.pallas_call(
    _final_body,
    grid=(GRID_N,),
    in_specs=[_P_SPEC, _X_SPEC, _X_SPEC,
              _W_SPEC, _B_SPEC, _W_SPEC, _B_SPEC, _W_SPEC, _B_SPEC,
              pl.BlockSpec((D, 1), lambda i: (0, 0)),
              pl.BlockSpec(memory_space=pltpu.SMEM)],
    out_specs=pl.BlockSpec(memory_space=pltpu.SMEM),
    out_shape=jax.ShapeDtypeStruct((1, 1), _f32),
)


def _sc_msg_body(h_hbm, filt_hbm, edges_hbm, out_hbm,
                 pk2, sidx, didx4, filt2, hrow3, agg_sh,
                 sem_e0, sem_e1, sem_f0, sem_f1, sem_g0, sem_g1,
                 sem_s0, sem_s1):
    c = lax.axis_index("c")
    s = lax.axis_index("s")
    wid = c * NS + s
    kw = lax.select(c == 0, K0, K1)
    coff = lax.select(c == 0, s * K0, NS * K0 + s * K1)
    sem_e = (sem_e0, sem_e1)
    sem_f = (sem_f0, sem_f1)
    sem_g = (sem_g0, sem_g1)
    sem_s = (sem_s0, sem_s1)

    zv = jnp.zeros((16,), _f32)

    def zrow(j, cc):
        for q in range(8):
            hrow3[0, j, pl.ds(q * 16, 16)] = zv
        return cc

    lax.fori_loop(0, CH, zrow, 0)
    for t in range(ROWS_SUB // CH):
        pltpu.sync_copy(hrow3.at[0], agg_sh.at[pl.ds(s * ROWS_SUB + t * CH, CH)])
    plsc.subcore_barrier()

    def unpack(b, jj, m4):
        for q in range(QN):
            sl = pl.ds(q * 16, 16)
            p = pk2[b, sl]
            sidx[b, sl] = jax.lax.shift_right_logical(p, 14)
            didx4[m4, sl] = jnp.bitwise_and(p, 16383)

    def issue_pk(b, jj):
        pltpu.async_copy(edges_hbm.at[wid, jj], pk2.at[b], sem_e[b])

    def issue_fg(b, jj, m3):
        base = (coff + jj) * CH
        pltpu.async_copy(filt_hbm.at[pl.ds(base, CH)], filt2.at[b], sem_f[b])
        pltpu.async_copy(h_hbm.at[sidx.at[b]], hrow3.at[m3], sem_g[b])

    for b in range(2):
        pltpu.sync_copy(edges_hbm.at[wid, b], pk2.at[b])
        unpack(b, b, b)
        issue_fg(b, b, b)
    for b in range(2):
        issue_pk(b, 2 + b)

    def drain_s(sem):
        pltpu.make_async_copy(hrow3.at[0], agg_sh.at[pl.ds(0, CH)], sem).wait()

    def pair(p, carry):
        for b in range(2):
            jj = 2 * p + b
            m3 = lax.rem(jj, 3)
            base = (coff + jj) * CH

            pltpu.make_async_copy(
                filt_hbm.at[pl.ds(base, CH)], filt2.at[b], sem_f[b]).wait()
            pltpu.make_async_copy(
                h_hbm.at[sidx.at[b]], hrow3.at[m3], sem_g[b]).wait()

            def row(rr, cc):
                for q in range(4):
                    w = filt2[b, rr, pl.ds(q * 16, 16)]
                    fe = jax.lax.bitcast_convert_type(jax.lax.shift_left(w, 16), _f32)
                    fo = jax.lax.bitcast_convert_type(jnp.bitwise_and(w, -65536), _f32)
                    sl_e = pl.ds(q * 16, 16)
                    sl_o = pl.ds(64 + q * 16, 16)
                    hrow3[m3, rr, sl_e] = hrow3[m3, rr, sl_e] * fe
                    hrow3[m3, rr, sl_o] = hrow3[m3, rr, sl_o] * fo
                return cc

            lax.fori_loop(0, CH, row, 0, unroll=4)
            pltpu.async_copy(hrow3.at[m3], agg_sh.at[didx4.at[lax.rem(jj, 4)]],
                             sem_s[b], add=True)

            @pl.when(jj >= 1)
            def _():
                drain_s(sem_s[1 - b])

            @pl.when(jj + 2 < kw)
            def _():
                pltpu.make_async_copy(
                    edges_hbm.at[wid, jj + 2], pk2.at[b], sem_e[b]).wait()
                unpack(b, jj + 2, lax.rem(jj + 2, 4))
                issue_fg(b, jj + 2, lax.rem(jj + 2, 3))

            @pl.when(jj + 4 < kw)
            def _():
                issue_pk(b, jj + 4)
        return carry

    lax.fori_loop(0, lax.div(kw, 2), pair, 0)
    drain_s(sem_s[1])

    plsc.subcore_barrier()
    pltpu.sync_copy(agg_sh.at[pl.ds(s * ROWS_SUB, ROWS_SUB)],
                    out_hbm.at[c, pl.ds(s * ROWS_SUB, ROWS_SUB)])


@functools.cache
def _sc_msg_call():
    return pl.kernel(
        _sc_msg_body,
        out_type=jax.ShapeDtypeStruct((NC, N_PAD, D), _f32),
        mesh=plsc.VectorSubcoreMesh(core_axis_name="c", subcore_axis_name="s",
                                    num_cores=NC, num_subcores=NS),
        scratch_types=[
            pltpu.VMEM((2, CH), jnp.int32),
            pltpu.VMEM((2, CH), jnp.int32),
            pltpu.VMEM((4, CH), jnp.int32),
            pltpu.VMEM((2, CH, D // 2), jnp.int32),
            pltpu.VMEM((3, CH, D), _f32),
            pltpu.VMEM_SHARED((N_PAD, D), _f32),
            pltpu.SemaphoreType.DMA,
            pltpu.SemaphoreType.DMA,
            pltpu.SemaphoreType.DMA,
            pltpu.SemaphoreType.DMA,
            pltpu.SemaphoreType.DMA,
            pltpu.SemaphoreType.DMA,
            pltpu.SemaphoreType.DMA,
            pltpu.SemaphoreType.DMA,
        ],
    )


def kernel(atomic_number, edge_index, r, atom_emb, pre_w, pre_b, filt_w, filt_b,
           post_w, post_b, ff_w1, ff_b1, ff_w2, ff_b2, fc_w, fc_b):
    an = atomic_number.astype(_f32).reshape(N_NODES, 1)
    emb_p = jnp.pad(atom_emb, ((0, 128 - atom_emb.shape[0]), (0, 0)))
    pre_wT = jnp.transpose(pre_w, (0, 2, 1))
    post_wT = jnp.transpose(post_w, (0, 2, 1))
    ff_w1T = jnp.transpose(ff_w1, (0, 2, 1))
    ff_w2T = jnp.transpose(ff_w2, (0, 2, 1))
    filt_wcat = jnp.concatenate([filt_w[i].T for i in range(L)], axis=1)
    filt_bcat = filt_b.reshape(1, L * D)
    fc_wT = fc_w.T
    fc_b2d = fc_b.reshape(1, 1)

    pad = E_PAD - N_EDGES
    packed = (edge_index[0].astype(jnp.int32) * 16384
              + edge_index[1].astype(jnp.int32))
    kmax = max(K0, K1)
    flat = jnp.pad(packed, (0, pad))
    c0 = flat[:NS * K0 * CH].reshape(NS, K0, CH)
    c0 = jnp.pad(c0, ((0, 0), (0, kmax - K0), (0, 0)))
    c1 = flat[NS * K0 * CH:].reshape(NS, K1, CH)
    c1 = jnp.pad(c1, ((0, 0), (0, kmax - K1), (0, 0)))
    edges3 = jnp.concatenate([c0, c1], axis=0)
    r8 = jnp.pad(r, ((0, pad), (0, 5)))

    x0, h = _embed_call(an, emb_p, pre_wT[0], pre_b[0].reshape(1, D))
    filts = _filt_call(r8, filt_wcat, filt_bcat)

    x = x0
    out = None
    for i in range(L):
        p = _sc_msg_call()(h, filts[i], edges3)
        if i < L - 1:
            x, h = _layer_call(p, x, x0,
                               post_wT[i], post_b[i].reshape(1, D),
                               ff_w1T[i], ff_b1[i].reshape(1, D),
                               ff_w2T[i], ff_b2[i].reshape(1, D),
                               pre_wT[i + 1], pre_b[i + 1].reshape(1, D))
        else:
            out = _final_call(p, x, x0,
                              post_wT[i], post_b[i].reshape(1, D),
                              ff_w1T[i], ff_b1[i].reshape(1, D),
                              ff_w2T[i], ff_b2[i].reshape(1, D),
                              fc_wT, fc_b2d)
    return out[0, 0]


# speedup vs baseline: 2.4459x; 1.0001x over previous
import functools

import jax
import jax.numpy as jnp
from jax import lax
from jax.experimental import pallas as pl
from jax.experimental.pallas import tpu as pltpu
from jax.experimental.pallas import tpu_sc as plsc

N_NODES = 10000
N_EDGES = 320000
D = 128
L = 4
R_CUTOFF = 8.0

NC = 2
NS = 16
NW = NC * NS
CH = 64
K = (-(-N_EDGES // (NW * CH)) + 7) // 8 * 8
K0 = K
K1 = 2 * K - K0
E_PAD = NS * (K0 + K1) * CH
N_PAD = 10240
ROWS_SUB = N_PAD // NS
CBYTES = CH * D * 4
QN = CH // 16

NBLK = 1000
GRID_N = N_NODES // NBLK
EBLK = 1024
GRID_E = E_PAD // EBLK

_f32 = jnp.float32


def _embed_body(an_ref, emb_ref, w_ref, b_ref, x0_ref, h0_ref):
    an = an_ref[...]
    ids = lax.broadcasted_iota(jnp.int32, (1, 128), 1).astype(_f32)
    oh = (an == ids).astype(_f32)
    x0 = jnp.dot(oh, emb_ref[...], preferred_element_type=_f32)
    x0_ref[...] = x0
    h0_ref[...] = jnp.dot(x0, w_ref[...], preferred_element_type=_f32) + b_ref[...]


_embed_call = pl.pallas_call(
    _embed_body,
    grid=(GRID_N,),
    in_specs=[
        pl.BlockSpec((NBLK, 1), lambda i: (i, 0)),
        pl.BlockSpec((128, D), lambda i: (0, 0)),
        pl.BlockSpec((D, D), lambda i: (0, 0)),
        pl.BlockSpec((1, D), lambda i: (0, 0)),
    ],
    out_specs=[
        pl.BlockSpec((NBLK, D), lambda i: (i, 0)),
        pl.BlockSpec((NBLK, D), lambda i: (i, 0)),
    ],
    out_shape=[
        jax.ShapeDtypeStruct((N_NODES, D), _f32),
        jax.ShapeDtypeStruct((N_NODES, D), _f32),
    ],
)


def _filt_body(r_ref, w_ref, b_ref, o0, o1, o2, o3):
    i = pl.program_id(0)
    r = r_ref[...]
    d = jnp.sqrt(jnp.sum(r * r, axis=1, keepdims=True))
    step = R_CUTOFF / (D - 1)
    centers = lax.broadcasted_iota(jnp.int32, (1, D), 1).astype(_f32) * step
    gamma = 1.0 / (step * step)
    rbf = jnp.exp(-gamma * (d - centers) ** 2)
    f = jnp.dot(rbf, w_ref[...], preferred_element_type=_f32) + b_ref[...]
    rowid = i * EBLK + lax.broadcasted_iota(jnp.int32, (EBLK, 1), 0)
    f = jnp.where(rowid < N_EDGES, f, 0.0)
    for l, o in enumerate((o0, o1, o2, o3)):
        fl = f[:, l * D:(l + 1) * D]
        bits = jax.lax.bitcast_convert_type(fl, jnp.int32)
        r16 = jax.lax.shift_right_logical(bits + 0x8000, 16)
        lo = jnp.bitwise_and(r16[:, :D // 2], 0xFFFF)
        hi = jax.lax.shift_left(r16[:, D // 2:], 16)
        o[...] = jnp.bitwise_or(lo, hi)


_filt_call = pl.pallas_call(
    _filt_body,
    grid=(GRID_E,),
    in_specs=[
        pl.BlockSpec((EBLK, 8), lambda i: (i, 0)),
        pl.BlockSpec((D, L * D), lambda i: (0, 0)),
        pl.BlockSpec((1, L * D), lambda i: (0, 0)),
    ],
    out_specs=[pl.BlockSpec((EBLK, D // 2), lambda i: (i, 0)) for _ in range(L)],
    out_shape=[jax.ShapeDtypeStruct((E_PAD, D // 2), jnp.int32) for _ in range(L)],
)


def _silu(x):
    return x * jax.nn.sigmoid(x)


def _layer_tail(p_ref, x_ref, x0_ref, postw, postb, f1w, f1b, f2w, f2b):
    agg = p_ref[0] + p_ref[1]
    t = jnp.dot(agg, postw[...], preferred_element_type=_f32) + postb[...]
    x1 = _silu(t) + x_ref[...]
    u = _silu(jnp.dot(x1, f1w[...], preferred_element_type=_f32) + f1b[...])
    ff = jnp.dot(u, f2w[...], preferred_element_type=_f32) + f2b[...]
    return ff + x1 + x0_ref[...]


def _layer_body(p_ref, x_ref, x0_ref, postw, postb, f1w, f1b, f2w, f2b,
                prew, preb, xo_ref, ho_ref):
    x2 = _layer_tail(p_ref, x_ref, x0_ref, postw, postb, f1w, f1b, f2w, f2b)
    xo_ref[...] = x2
    ho_ref[...] = jnp.dot(x2, prew[...], preferred_element_type=_f32) + preb[...]


_W_SPEC = pl.BlockSpec((D, D), lambda i: (0, 0))
_B_SPEC = pl.BlockSpec((1, D), lambda i: (0, 0))
_X_SPEC = pl.BlockSpec((NBLK, D), lambda i: (i, 0))
_P_SPEC = pl.BlockSpec((NC, NBLK, D), lambda i: (0, i, 0))

_layer_call = pl.pallas_call(
    _layer_body,
    grid=(GRID_N,),
    in_specs=[_P_SPEC, _X_SPEC, _X_SPEC,
              _W_SPEC, _B_SPEC, _W_SPEC, _B_SPEC, _W_SPEC, _B_SPEC,
              _W_SPEC, _B_SPEC],
    out_specs=[_X_SPEC, _X_SPEC],
    out_shape=[
        jax.ShapeDtypeStruct((N_NODES, D), _f32),
        jax.ShapeDtypeStruct((N_NODES, D), _f32),
    ],
)


def _final_body(p_ref, x_ref, x0_ref, postw, postb, f1w, f1b, f2w, f2b,
                fcw_ref, fcb_ref, out_ref):
    i = pl.program_id(0)
    x2 = _layer_tail(p_ref, x_ref, x0_ref, postw, postb, f1w, f1b, f2w, f2b)
    e = jnp.dot(x2, fcw_ref[...], preferred_element_type=_f32)

    @pl.when(i == 0)
    def _():
        out_ref[0, 0] = 0.0

    out_ref[0, 0] += jnp.sum(e)

    @pl.when(i == GRID_N - 1)
    def _():
        out_ref[0, 0] = out_ref[0, 0] / N_NODES + fcb_ref[0, 0]


_final_call = pl.pallas_call(
    _final_body,
    grid=(GRID_N,),
    in_specs=[_P_SPEC, _X_SPEC, _X_SPEC,
              _W_SPEC, _B_SPEC, _W_SPEC, _B_SPEC, _W_SPEC, _B_SPEC,
              pl.BlockSpec((D, 1), lambda i: (0, 0)),
              pl.BlockSpec(memory_space=pltpu.SMEM)],
    out_specs=pl.BlockSpec(memory_space=pltpu.SMEM),
    out_shape=jax.ShapeDtypeStruct((1, 1), _f32),
)


def _sc_msg_body(h_hbm, filt_hbm, edges_hbm, out_hbm,
                 pk2, sidx, didx4, filt2, hrow3, agg_sh,
                 sem_e0, sem_e1, sem_f0, sem_f1, sem_g0, sem_g1,
                 sem_s0, sem_s1):
    c = lax.axis_index("c")
    s = lax.axis_index("s")
    wid = c * NS + s
    kw = K
    coff = wid * K
    sem_e = (sem_e0, sem_e1)
    sem_f = (sem_f0, sem_f1)
    sem_g = (sem_g0, sem_g1)
    sem_s = (sem_s0, sem_s1)

    zv = jnp.zeros((16,), _f32)

    def zrow(j, cc):
        for q in range(8):
            hrow3[0, j, pl.ds(q * 16, 16)] = zv
        return cc

    lax.fori_loop(0, CH, zrow, 0)
    for t in range(ROWS_SUB // CH):
        pltpu.sync_copy(hrow3.at[0], agg_sh.at[pl.ds(s * ROWS_SUB + t * CH, CH)])
    plsc.subcore_barrier()

    def unpack(b, jj, m4):
        for q in range(QN):
            sl = pl.ds(q * 16, 16)
            p = pk2[b, sl]
            sidx[b, sl] = jax.lax.shift_right_logical(p, 14)
            didx4[m4, sl] = jnp.bitwise_and(p, 16383)

    def issue_pk(b, jj):
        pltpu.async_copy(edges_hbm.at[wid, jj], pk2.at[b], sem_e[b])

    def issue_fg(b, jj, m3):
        base = (coff + jj) * CH
        pltpu.async_copy(filt_hbm.at[pl.ds(base, CH)], filt2.at[b], sem_f[b])
        pltpu.async_copy(h_hbm.at[sidx.at[b]], hrow3.at[m3], sem_g[b])

    for b in range(2):
        pltpu.sync_copy(edges_hbm.at[wid, b], pk2.at[b])
        unpack(b, b, b)
        issue_fg(b, b, b)
    for b in range(2):
        issue_pk(b, 2 + b)

    def drain_s(sem):
        pltpu.make_async_copy(hrow3.at[0], agg_sh.at[pl.ds(0, CH)], sem).wait()

    def pair(p, carry):
        for b in range(2):
            jj = 2 * p + b
            m3 = lax.rem(jj, 3)
            base = (coff + jj) * CH

            pltpu.make_async_copy(
                filt_hbm.at[pl.ds(base, CH)], filt2.at[b], sem_f[b]).wait()
            pltpu.make_async_copy(
                h_hbm.at[sidx.at[b]], hrow3.at[m3], sem_g[b]).wait()

            def row(rr, cc):
                for q in range(4):
                    w = filt2[b, rr, pl.ds(q * 16, 16)]
                    fe = jax.lax.bitcast_convert_type(jax.lax.shift_left(w, 16), _f32)
                    fo = jax.lax.bitcast_convert_type(jnp.bitwise_and(w, -65536), _f32)
                    sl_e = pl.ds(q * 16, 16)
                    sl_o = pl.ds(64 + q * 16, 16)
                    hrow3[m3, rr, sl_e] = hrow3[m3, rr, sl_e] * fe
                    hrow3[m3, rr, sl_o] = hrow3[m3, rr, sl_o] * fo
                return cc

            lax.fori_loop(0, CH, row, 0, unroll=4)
            pltpu.async_copy(hrow3.at[m3], agg_sh.at[didx4.at[lax.rem(jj, 4)]],
                             sem_s[b], add=True)

            @pl.when(jj >= 1)
            def _():
                drain_s(sem_s[1 - b])

            @pl.when(jj + 2 < kw)
            def _():
                pltpu.make_async_copy(
                    edges_hbm.at[wid, jj + 2], pk2.at[b], sem_e[b]).wait()
                unpack(b, jj + 2, lax.rem(jj + 2, 4))
                issue_fg(b, jj + 2, lax.rem(jj + 2, 3))

            @pl.when(jj + 4 < kw)
            def _():
                issue_pk(b, jj + 4)
        return carry

    lax.fori_loop(0, K // 2, pair, 0)
    drain_s(sem_s[1])

    plsc.subcore_barrier()
    pltpu.sync_copy(agg_sh.at[pl.ds(s * ROWS_SUB, ROWS_SUB)],
                    out_hbm.at[c, pl.ds(s * ROWS_SUB, ROWS_SUB)])


@functools.cache
def _sc_msg_call():
    return pl.kernel(
        _sc_msg_body,
        out_type=jax.ShapeDtypeStruct((NC, N_PAD, D), _f32),
        mesh=plsc.VectorSubcoreMesh(core_axis_name="c", subcore_axis_name="s",
                                    num_cores=NC, num_subcores=NS),
        scratch_types=[
            pltpu.VMEM((2, CH), jnp.int32),
            pltpu.VMEM((2, CH), jnp.int32),
            pltpu.VMEM((4, CH), jnp.int32),
            pltpu.VMEM((2, CH, D // 2), jnp.int32),
            pltpu.VMEM((3, CH, D), _f32),
            pltpu.VMEM_SHARED((N_PAD, D), _f32),
            pltpu.SemaphoreType.DMA,
            pltpu.SemaphoreType.DMA,
            pltpu.SemaphoreType.DMA,
            pltpu.SemaphoreType.DMA,
            pltpu.SemaphoreType.DMA,
            pltpu.SemaphoreType.DMA,
            pltpu.SemaphoreType.DMA,
            pltpu.SemaphoreType.DMA,
        ],
    )


def kernel(atomic_number, edge_index, r, atom_emb, pre_w, pre_b, filt_w, filt_b,
           post_w, post_b, ff_w1, ff_b1, ff_w2, ff_b2, fc_w, fc_b):
    an = atomic_number.astype(_f32).reshape(N_NODES, 1)
    emb_p = jnp.pad(atom_emb, ((0, 128 - atom_emb.shape[0]), (0, 0)))
    pre_wT = jnp.transpose(pre_w, (0, 2, 1))
    post_wT = jnp.transpose(post_w, (0, 2, 1))
    ff_w1T = jnp.transpose(ff_w1, (0, 2, 1))
    ff_w2T = jnp.transpose(ff_w2, (0, 2, 1))
    filt_wcat = jnp.concatenate([filt_w[i].T for i in range(L)], axis=1)
    filt_bcat = filt_b.reshape(1, L * D)
    fc_wT = fc_w.T
    fc_b2d = fc_b.reshape(1, 1)

    pad = E_PAD - N_EDGES
    packed = (edge_index[0].astype(jnp.int32) * 16384
              + edge_index[1].astype(jnp.int32))
    kmax = max(K0, K1)
    flat = jnp.pad(packed, (0, pad))
    c0 = flat[:NS * K0 * CH].reshape(NS, K0, CH)
    c0 = jnp.pad(c0, ((0, 0), (0, kmax - K0), (0, 0)))
    c1 = flat[NS * K0 * CH:].reshape(NS, K1, CH)
    c1 = jnp.pad(c1, ((0, 0), (0, kmax - K1), (0, 0)))
    edges3 = jnp.concatenate([c0, c1], axis=0)
    r8 = jnp.pad(r, ((0, pad), (0, 5)))

    x0, h = _embed_call(an, emb_p, pre_wT[0], pre_b[0].reshape(1, D))
    filts = _filt_call(r8, filt_wcat, filt_bcat)

    x = x0
    out = None
    for i in range(L):
        p = _sc_msg_call()(h, filts[i], edges3)
        if i < L - 1:
            x, h = _layer_call(p, x, x0,
                               post_wT[i], post_b[i].reshape(1, D),
                               ff_w1T[i], ff_b1[i].reshape(1, D),
                               ff_w2T[i], ff_b2[i].reshape(1, D),
                               pre_wT[i + 1], pre_b[i + 1].reshape(1, D))
        else:
            out = _final_call(p, x, x0,
                              post_wT[i], post_b[i].reshape(1, D),
                              ff_w1T[i], ff_b1[i].reshape(1, D),
                              ff_w2T[i], ff_b2[i].reshape(1, D),
                              fc_wT, fc_b2d)
    return out[0, 0]
